# baseline (speedup 1.0000x reference)
import functools

import jax
import jax.numpy as jnp
from jax import lax
from jax.experimental import pallas as pl
from jax.experimental.pallas import tpu as pltpu
from jax.experimental.pallas import tpu_sc as plsc

_B, _S, _G = 4, 2048, 4
_VOCAB, _DIM = 100000, 128
_NC, _NS = 2, 16
_NW = _NC * _NS
_ROWS = _B * _S
_RPW = _ROWS // _NW
_GPW = _RPW * _G
_CHUNK = 128
_NCHUNK = _GPW // _CHUNK
_OPC = _CHUNK // _G

_mesh = plsc.VectorSubcoreMesh(core_axis_name="c", subcore_axis_name="s")


@functools.partial(
    pl.kernel,
    mesh=_mesh,
    out_type=jax.ShapeDtypeStruct((_ROWS, _DIM), jnp.float32),
    scratch_types=[
        pltpu.VMEM((_GPW,), jnp.int32),
        pltpu.VMEM((_CHUNK, _DIM), jnp.float32),
        pltpu.VMEM((_CHUNK, _DIM), jnp.float32),
        pltpu.VMEM((_CHUNK, _DIM), jnp.float32),
        pltpu.VMEM((_CHUNK, _DIM), jnp.float32),
        pltpu.VMEM((_RPW, _DIM), jnp.float32),
        pltpu.SemaphoreType.DMA,
        pltpu.SemaphoreType.DMA,
        pltpu.SemaphoreType.DMA,
        pltpu.SemaphoreType.DMA,
    ],
)
def _embed_sum(x_hbm, tab_hbm, out_hbm, idx_v, rows_0, rows_1, rows_2, rows_3,
               out_v, sem_0, sem_1, sem_2, sem_3):
    wid = lax.axis_index("s") * _NC + lax.axis_index("c")
    pltpu.sync_copy(x_hbm.at[pl.ds(wid * _GPW, _GPW)], idx_v)

    off = (lax.iota(jnp.int32, 16) % _G) * _VOCAB
    for i in range(_GPW // 16):
        sl = pl.ds(i * 16, 16)
        idx_v[sl] = idx_v[sl] + off

    bufs = (rows_0, rows_1, rows_2, rows_3)
    sems = (sem_0, sem_1, sem_2, sem_3)
    _NBUF = 4
    _RU = 4

    def start(j):
        return pltpu.async_copy(
            tab_hbm.at[idx_v.at[pl.ds(j * _CHUNK, _CHUNK)]],
            bufs[j % _NBUF],
            sems[j % _NBUF],
        )

    copies = [start(j) for j in range(_NBUF - 1)]
    for j in range(_NCHUNK):
        if j + _NBUF - 1 < _NCHUNK:
            copies.append(start(j + _NBUF - 1))
        copies.pop(0).wait()
        buf = bufs[j % _NBUF]

        def row_body(i, carry, j=j, buf=buf):
            r = i * _RU
            for u in range(_RU):
                for c in range(_DIM // 16):
                    sl = pl.ds(c * 16, 16)
                    v = (buf[4 * (r + u), sl] + buf[4 * (r + u) + 1, sl]) + (
                        buf[4 * (r + u) + 2, sl] + buf[4 * (r + u) + 3, sl]
                    )
                    out_v[j * _OPC + r + u, sl] = v
            return carry

        lax.fori_loop(0, _OPC // _RU, row_body, 0)

    pltpu.sync_copy(out_v, out_hbm.at[pl.ds(wid * _RPW, _RPW)])


def kernel(x, tables):
    xf = x.reshape(_ROWS * _G)
    tf = tables.reshape(_G * _VOCAB, _DIM)
    out = _embed_sum(xf, tf)
    return out.reshape(_B, _S, _DIM)


# speedup vs baseline: 1.0324x; 1.0324x over previous
import functools

import jax
import jax.numpy as jnp
from jax import lax
from jax.experimental import pallas as pl
from jax.experimental.pallas import tpu as pltpu
from jax.experimental.pallas import tpu_sc as plsc

_B, _S, _G = 4, 2048, 4
_VOCAB, _DIM = 100000, 128
_NC, _NS = 2, 16
_NW = _NC * _NS
_ROWS = _B * _S
_RPW = _ROWS // _NW
_GPW = _RPW * _G
_CHUNK = 128
_NCHUNK = _GPW // _CHUNK
_OPC = _CHUNK // _G

_mesh = plsc.VectorSubcoreMesh(core_axis_name="c", subcore_axis_name="s")


@functools.partial(
    pl.kernel,
    mesh=_mesh,
    out_type=jax.ShapeDtypeStruct((_ROWS, _DIM), jnp.float32),
    scratch_types=[
        pltpu.VMEM((_GPW,), jnp.int32),
        pltpu.VMEM((_CHUNK, _DIM), jnp.float32),
        pltpu.VMEM((_CHUNK, _DIM), jnp.float32),
        pltpu.VMEM((_CHUNK, _DIM), jnp.float32),
        pltpu.VMEM((_CHUNK, _DIM), jnp.float32),
        pltpu.VMEM((_RPW, _DIM), jnp.float32),
        pltpu.SemaphoreType.DMA,
        pltpu.SemaphoreType.DMA,
        pltpu.SemaphoreType.DMA,
        pltpu.SemaphoreType.DMA,
    ],
)
def _embed_sum(x_hbm, tab_hbm, out_hbm, idx_v, rows_0, rows_1, rows_2, rows_3,
               out_v, sem_0, sem_1, sem_2, sem_3):
    wid = lax.axis_index("s") * _NC + lax.axis_index("c")
    pltpu.sync_copy(x_hbm.at[pl.ds(wid * _GPW, _GPW)], idx_v)

    off = (lax.iota(jnp.int32, 16) % _G) * _VOCAB
    for i in range(_GPW // 16):
        sl = pl.ds(i * 16, 16)
        idx_v[sl] = idx_v[sl] + off

    bufs = (rows_0, rows_1)
    sems = (sem_0, sem_1)
    _NBUF = 2
    _RU = 4

    def start(j):
        return pltpu.async_copy(
            tab_hbm.at[idx_v.at[pl.ds(j * _CHUNK, _CHUNK)]],
            bufs[j % _NBUF],
            sems[j % _NBUF],
        )

    copies = [start(j) for j in range(_NBUF - 1)]
    for j in range(_NCHUNK):
        if j + _NBUF - 1 < _NCHUNK:
            copies.append(start(j + _NBUF - 1))
        copies.pop(0).wait()
        buf = bufs[j % _NBUF]

        def row_body(i, carry, j=j, buf=buf):
            r = i * _RU
            for u in range(_RU):
                for c in range(_DIM // 16):
                    sl = pl.ds(c * 16, 16)
                    v = (buf[4 * (r + u), sl] + buf[4 * (r + u) + 1, sl]) + (
                        buf[4 * (r + u) + 2, sl] + buf[4 * (r + u) + 3, sl]
                    )
                    out_v[j * _OPC + r + u, sl] = v
            return carry

        lax.fori_loop(0, _OPC // _RU, row_body, 0)

    pltpu.sync_copy(out_v, out_hbm.at[pl.ds(wid * _RPW, _RPW)])


def kernel(x, tables):
    xf = x.reshape(_ROWS * _G)
    tf = tables.reshape(_G * _VOCAB, _DIM)
    out = _embed_sum(xf, tf)
    return out.reshape(_B, _S, _DIM)


# speedup vs baseline: 1.0326x; 1.0002x over previous
import functools

import jax
import jax.numpy as jnp
from jax import lax
from jax.experimental import pallas as pl
from jax.experimental.pallas import tpu as pltpu
from jax.experimental.pallas import tpu_sc as plsc

_B, _S, _G = 4, 2048, 4
_VOCAB, _DIM = 100000, 128
_NC, _NS = 2, 16
_NW = _NC * _NS
_ROWS = _B * _S
_RPW = _ROWS // _NW
_GPW = _RPW * _G
_CHUNK = 128
_NCHUNK = _GPW // _CHUNK
_OPC = _CHUNK // _G

_mesh = plsc.VectorSubcoreMesh(core_axis_name="c", subcore_axis_name="s")


@functools.partial(
    pl.kernel,
    mesh=_mesh,
    out_type=jax.ShapeDtypeStruct((_ROWS, _DIM), jnp.float32),
    scratch_types=[
        pltpu.VMEM((_GPW,), jnp.int32),
        pltpu.VMEM((_CHUNK, _DIM), jnp.float32),
        pltpu.VMEM((_CHUNK, _DIM), jnp.float32),
        pltpu.VMEM((_CHUNK, _DIM), jnp.float32),
        pltpu.VMEM((_CHUNK, _DIM), jnp.float32),
        pltpu.VMEM((_RPW, _DIM), jnp.float32),
        pltpu.SemaphoreType.DMA,
        pltpu.SemaphoreType.DMA,
        pltpu.SemaphoreType.DMA,
        pltpu.SemaphoreType.DMA,
    ],
)
def _embed_sum(x_hbm, tab_hbm, out_hbm, idx_v, rows_0, rows_1, rows_2, rows_3,
               out_v, sem_0, sem_1, sem_2, sem_3):
    wid = lax.axis_index("s") * _NC + lax.axis_index("c")
    with jax.named_scope("idx_load"):
        pltpu.sync_copy(x_hbm.at[pl.ds(wid * _GPW, _GPW)], idx_v)

    with jax.named_scope("idx_offset"):
        off = (lax.iota(jnp.int32, 16) % _G) * _VOCAB
        for i in range(_GPW // 16):
            sl = pl.ds(i * 16, 16)
            idx_v[sl] = idx_v[sl] + off

    bufs = (rows_0, rows_1)
    sems = (sem_0, sem_1)
    _NBUF = 2
    _RU = 4

    def start(j):
        return pltpu.async_copy(
            tab_hbm.at[idx_v.at[pl.ds(j * _CHUNK, _CHUNK)]],
            bufs[j % _NBUF],
            sems[j % _NBUF],
        )

    copies = [start(j) for j in range(_NBUF - 1)]
    for j in range(_NCHUNK):
        if j + _NBUF - 1 < _NCHUNK:
            copies.append(start(j + _NBUF - 1))
        with jax.named_scope(f"wait{j}"):
            copies.pop(0).wait()
        buf = bufs[j % _NBUF]

        def row_body(i, carry, j=j, buf=buf):
            r = i * _RU
            for u in range(_RU):
                for c in range(_DIM // 16):
                    sl = pl.ds(c * 16, 16)
                    v = (buf[4 * (r + u), sl] + buf[4 * (r + u) + 1, sl]) + (
                        buf[4 * (r + u) + 2, sl] + buf[4 * (r + u) + 3, sl]
                    )
                    out_v[j * _OPC + r + u, sl] = v
            return carry

        with jax.named_scope(f"sum{j}"):
            lax.fori_loop(0, _OPC // _RU, row_body, 0)

    with jax.named_scope("out_store"):
        pltpu.sync_copy(out_v, out_hbm.at[pl.ds(wid * _RPW, _RPW)])


def kernel(x, tables):
    xf = x.reshape(_ROWS * _G)
    tf = tables.reshape(_G * _VOCAB, _DIM)
    out = _embed_sum(xf, tf)
    return out.reshape(_B, _S, _DIM)
